# baseline (speedup 1.0000x reference)
import functools

import jax
import jax.numpy as jnp
from jax.experimental import pallas as pl

_TI = 128
_TJ = 128
_D = 128


def _emd_tile(yi_ref, yjt_ref, x_ref, out_ref):
    i0 = pl.program_id(0)
    j0 = pl.program_id(1)

    @pl.when((i0 == 0) & (j0 == 0))
    def _init():
        out_ref[...] = jnp.zeros_like(out_ref)

    yi = yi_ref[...]
    yjt = yjt_ref[...]

    acc = jnp.zeros((_TI, _TJ), jnp.float32)
    for d in range(_D):
        acc = acc + jnp.abs(yi[:, d : d + 1] - yjt[d : d + 1, :])
    out_ref[...] += jnp.sum(acc * x_ref[...]).reshape(1, 1)


@jax.jit
def kernel(y_i, y_j, x_ij):
    n_i, d = y_i.shape
    n_j = y_j.shape[0]
    yjt = y_j.T

    out = pl.pallas_call(
        _emd_tile,
        grid=(n_i // _TI, n_j // _TJ),
        in_specs=[
            pl.BlockSpec((_TI, d), lambda i, j: (i, 0)),
            pl.BlockSpec((d, _TJ), lambda i, j: (0, j)),
            pl.BlockSpec((_TI, _TJ), lambda i, j: (i, j)),
        ],
        out_specs=pl.BlockSpec((1, 1), lambda i, j: (0, 0)),
        out_shape=jax.ShapeDtypeStruct((1, 1), jnp.float32),
    )(y_i, yjt, x_ij)
    return out[0, 0]


# speedup vs baseline: 2.2839x; 2.2839x over previous
import jax
import jax.numpy as jnp
from jax.experimental import pallas as pl
from jax.experimental.pallas import tpu as pltpu

_TI = 128
_TJ = 128
_D = 128


def _emd_tile(yi_ref, yjt_ref, x_ref, out_ref, bc_ref):
    i0 = pl.program_id(0)
    j0 = pl.program_id(1)

    @pl.when((i0 == 0) & (j0 == 0))
    def _init():
        out_ref[...] = jnp.zeros_like(out_ref)

    @pl.when(j0 == 0)
    def _build():
        yi = yi_ref[...]
        for d in range(_D):
            bc_ref[pl.ds(d * _TI, _TI), :] = jnp.broadcast_to(
                yi[:, d : d + 1], (_TI, _TJ)
            )

    yjt = yjt_ref[...]
    acc = jnp.zeros((_TI, _TJ), jnp.float32)
    for d in range(_D):
        acc = acc + jnp.abs(bc_ref[pl.ds(d * _TI, _TI), :] - yjt[d : d + 1, :])
    out_ref[...] += jnp.sum(acc * x_ref[...]).reshape(1, 1)


@jax.jit
def kernel(y_i, y_j, x_ij):
    n_i, d = y_i.shape
    n_j = y_j.shape[0]
    yjt = y_j.T

    out = pl.pallas_call(
        _emd_tile,
        grid=(n_i // _TI, n_j // _TJ),
        in_specs=[
            pl.BlockSpec((_TI, d), lambda i, j: (i, 0)),
            pl.BlockSpec((d, _TJ), lambda i, j: (0, j)),
            pl.BlockSpec((_TI, _TJ), lambda i, j: (i, j)),
        ],
        out_specs=pl.BlockSpec((1, 1), lambda i, j: (0, 0)),
        out_shape=jax.ShapeDtypeStruct((1, 1), jnp.float32),
        scratch_shapes=[pltpu.VMEM((_D * _TI, _TJ), jnp.float32)],
    )(y_i, yjt, x_ij)
    return out[0, 0]


# speedup vs baseline: 2.5947x; 1.1361x over previous
import jax
import jax.numpy as jnp
from jax.experimental import pallas as pl
from jax.experimental.pallas import tpu as pltpu

_TI = 128
_TJ = 512
_H = _TJ // 128
_D = 128


def _emd_tile(yi_ref, yjt_ref, x_ref, out_ref, bc_ref):
    i0 = pl.program_id(0)
    j0 = pl.program_id(1)

    @pl.when((i0 == 0) & (j0 == 0))
    def _init():
        out_ref[...] = jnp.zeros_like(out_ref)

    @pl.when(j0 == 0)
    def _build():
        yi = yi_ref[...]
        for d in range(_D):
            bc_ref[pl.ds(d * _TI, _TI), :] = jnp.broadcast_to(
                yi[:, d : d + 1], (_TI, 128)
            )

    yjt = yjt_ref[...]
    x = x_ref[...]
    part = jnp.zeros((8, 128), jnp.float32)
    for h in range(_H):
        acc = jnp.zeros((_TI, 128), jnp.float32)
        for d in range(_D):
            acc = acc + jnp.abs(
                bc_ref[pl.ds(d * _TI, _TI), :]
                - yjt[d : d + 1, h * 128 : (h + 1) * 128]
            )
        w = acc * x[:, h * 128 : (h + 1) * 128]
        part = part + w.reshape(16, 8, 128).sum(axis=0)
    out_ref[...] += jnp.sum(part).reshape(1, 1)


@jax.jit
def kernel(y_i, y_j, x_ij):
    n_i, d = y_i.shape
    n_j = y_j.shape[0]
    yjt = y_j.T

    out = pl.pallas_call(
        _emd_tile,
        grid=(n_i // _TI, n_j // _TJ),
        in_specs=[
            pl.BlockSpec((_TI, d), lambda i, j: (i, 0)),
            pl.BlockSpec((d, _TJ), lambda i, j: (0, j)),
            pl.BlockSpec((_TI, _TJ), lambda i, j: (i, j)),
        ],
        out_specs=pl.BlockSpec((1, 1), lambda i, j: (0, 0)),
        out_shape=jax.ShapeDtypeStruct((1, 1), jnp.float32),
        scratch_shapes=[pltpu.VMEM((_D * _TI, 128), jnp.float32)],
    )(y_i, yjt, x_ij)
    return out[0, 0]
